# Initial kernel scaffold; baseline (speedup 1.0000x reference)
import jax
import jax.numpy as jnp
from jax.experimental import pallas as pl


def kernel(x, edge_index, batch, y, W1, b1, W2, b2, W3, b3, lin1_W, lin1_b, lin2_W, lin2_b):
    raise NotImplementedError("write your pallas kernel here")


# speedup vs baseline: 11.5456x; 11.5456x over previous
import functools

import jax
import jax.numpy as jnp
from jax import lax
from jax.experimental import pallas as pl
from jax.experimental.pallas import tpu as pltpu
from jax.experimental.pallas import tpu_sc as plsc

N = 10000
E = 160000
D = 256
NUM_CLASSES = 2

NCORES = 2
NSUB = 16
NWORK = NCORES * NSUB
C = 128
CH = 40
EW = C * CH
EPAD = EW * NWORK
NP = 10240
RPT = NP // NSUB


def _make_edge_pass(width):
  mesh = plsc.VectorSubcoreMesh(core_axis_name="c", subcore_axis_name="s")

  @functools.partial(
      pl.kernel,
      out_type=jax.ShapeDtypeStruct((NCORES, NP, width), jnp.float32),
      mesh=mesh,
      scratch_types=[
          pltpu.VMEM((CH, C), jnp.int32),
          pltpu.VMEM((CH, C), jnp.int32),
          pltpu.VMEM((C, width), jnp.float32),
          pltpu.VMEM_SHARED((NP, width), jnp.float32),
          pltpu.SemaphoreType.DMA,
      ],
      compiler_params=pltpu.CompilerParams(use_tc_tiling_on_sc=False),
  )
  def edge_pass(src_hbm, dst_hbm, table_hbm, zeros_hbm, out_hbm,
                src_v, dst_v, rows_v, acc_sh, sem):
    cid = lax.axis_index("c")
    sid = lax.axis_index("s")
    wid = cid * NSUB + sid
    pltpu.sync_copy(zeros_hbm.at[pl.ds(sid * RPT, RPT)],
                    acc_sh.at[pl.ds(sid * RPT, RPT)])
    pltpu.sync_copy(src_hbm.at[pl.ds(wid * CH, CH)], src_v)
    pltpu.sync_copy(dst_hbm.at[pl.ds(wid * CH, CH)], dst_v)
    plsc.subcore_barrier()

    def step(j, carry):
      pltpu.async_copy(table_hbm.at[src_v.at[j]], rows_v, sem).wait()
      pltpu.sync_copy(rows_v, acc_sh.at[dst_v.at[j]], add=True)
      return carry

    lax.fori_loop(0, CH, step, 0, unroll=False)
    plsc.subcore_barrier()
    pltpu.sync_copy(acc_sh.at[pl.ds(sid * RPT, RPT)],
                    out_hbm.at[cid, pl.ds(sid * RPT, RPT)])

  return edge_pass


_edge_pass_16 = _make_edge_pass(16)
_edge_pass_32 = _make_edge_pass(32)


BN = 1024


def _prep_body(deg_ref, x_ref, w1_ref, dinv_ref, s_ref):
  deg = deg_ref[0, :, 0:1] + deg_ref[1, :, 0:1] + 1.0
  dinv = 1.0 / jnp.sqrt(deg)
  dinv_ref[...] = dinv
  hw = jnp.dot(x_ref[...], w1_ref[...], preferred_element_type=jnp.float32)
  s_ref[...] = dinv * hw


def _tc_prep(degp, xpad, w1):
  return pl.pallas_call(
      _prep_body,
      grid=(NP // BN,),
      in_specs=[
          pl.BlockSpec((NCORES, BN, 16), lambda i: (0, i, 0)),
          pl.BlockSpec((BN, D), lambda i: (i, 0)),
          pl.BlockSpec((D, 32), lambda i: (0, 0)),
      ],
      out_specs=[
          pl.BlockSpec((BN, 1), lambda i: (i, 0)),
          pl.BlockSpec((BN, 32), lambda i: (i, 0)),
      ],
      out_shape=[
          jax.ShapeDtypeStruct((NP, 1), jnp.float32),
          jax.ShapeDtypeStruct((NP, 32), jnp.float32),
      ],
  )(degp, xpad, w1)


def _mid_body(p_ref, s_ref, dinv_ref, b_ref, wn_ref, h_ref, sn_ref):
  dinv = dinv_ref[...]
  h = jnp.tanh(dinv * (p_ref[0] + p_ref[1] + s_ref[...]) + b_ref[...])
  h_ref[...] = h
  sn_ref[...] = dinv * jnp.dot(h, wn_ref[...],
                               preferred_element_type=jnp.float32)


def _tc_mid(p, s, dinv, b, wn):
  return pl.pallas_call(
      _mid_body,
      grid=(NP // BN,),
      in_specs=[
          pl.BlockSpec((NCORES, BN, 32), lambda i: (0, i, 0)),
          pl.BlockSpec((BN, 32), lambda i: (i, 0)),
          pl.BlockSpec((BN, 1), lambda i: (i, 0)),
          pl.BlockSpec((1, 32), lambda i: (0, 0)),
          pl.BlockSpec((32, 32), lambda i: (0, 0)),
      ],
      out_specs=[
          pl.BlockSpec((BN, 32), lambda i: (i, 0)),
          pl.BlockSpec((BN, 32), lambda i: (i, 0)),
      ],
      out_shape=[
          jax.ShapeDtypeStruct((NP, 32), jnp.float32),
          jax.ShapeDtypeStruct((NP, 32), jnp.float32),
      ],
  )(p, s, dinv, b, wn)


BNF = 1000


def _final_body(p_ref, s_ref, dinv_ref, b_ref, h1_ref, h2_ref,
                l1w_ref, l1b_ref, l2w_ref, l2b_ref, y_ref,
                logits_ref, feat_ref, loss_ref, acc_ref):
  i = pl.program_id(0)
  dinv = dinv_ref[...]
  h3 = jnp.tanh(dinv * (p_ref[0] + p_ref[1] + s_ref[...]) + b_ref[...])
  cat = jnp.concatenate([h1_ref[...], h2_ref[...], h3], axis=1)
  hidden = jnp.dot(cat, l1w_ref[...],
                   preferred_element_type=jnp.float32) + l1b_ref[...]
  feat_ref[...] = hidden
  hr = jnp.maximum(hidden, 0.0)
  lg = jnp.dot(hr, l2w_ref[...],
               preferred_element_type=jnp.float32) + l2b_ref[...]
  m = jnp.max(lg, axis=1, keepdims=True)
  ls = lg - m - jnp.log(jnp.sum(jnp.exp(lg - m), axis=1, keepdims=True))
  logits_ref[...] = ls
  y2 = y_ref[...]
  l0 = ls[:, 0:1]
  l1 = ls[:, 1:2]
  picked = jnp.where(y2 == 0, l0, l1)
  pred = (l1 > l0).astype(jnp.int32)
  correct = (pred == y2).astype(jnp.float32)

  @pl.when(i == 0)
  def _():
    loss_ref[0, 0] = 0.0
    acc_ref[0, 0] = 0.0

  loss_ref[0, 0] += -jnp.sum(picked) / N
  acc_ref[0, 0] += jnp.sum(correct) / N


def _tc_final(p, s, dinv, b, h1, h2, l1w, l1b, l2w, l2b, y2):
  return pl.pallas_call(
      _final_body,
      grid=(N // BNF,),
      in_specs=[
          pl.BlockSpec((NCORES, BNF, 32), lambda i: (0, i, 0)),
          pl.BlockSpec((BNF, 32), lambda i: (i, 0)),
          pl.BlockSpec((BNF, 1), lambda i: (i, 0)),
          pl.BlockSpec((1, 32), lambda i: (0, 0)),
          pl.BlockSpec((BNF, 32), lambda i: (i, 0)),
          pl.BlockSpec((BNF, 32), lambda i: (i, 0)),
          pl.BlockSpec((96, 128), lambda i: (0, 0)),
          pl.BlockSpec((1, 128), lambda i: (0, 0)),
          pl.BlockSpec((128, NUM_CLASSES), lambda i: (0, 0)),
          pl.BlockSpec((1, NUM_CLASSES), lambda i: (0, 0)),
          pl.BlockSpec((BNF, 1), lambda i: (i, 0)),
      ],
      out_specs=[
          pl.BlockSpec((BNF, NUM_CLASSES), lambda i: (i, 0)),
          pl.BlockSpec((BNF, 128), lambda i: (i, 0)),
          pl.BlockSpec((1, 1), lambda i: (0, 0), memory_space=pltpu.SMEM),
          pl.BlockSpec((1, 1), lambda i: (0, 0), memory_space=pltpu.SMEM),
      ],
      out_shape=[
          jax.ShapeDtypeStruct((N, NUM_CLASSES), jnp.float32),
          jax.ShapeDtypeStruct((N, 128), jnp.float32),
          jax.ShapeDtypeStruct((1, 1), jnp.float32),
          jax.ShapeDtypeStruct((1, 1), jnp.float32),
      ],
  )(p, s, dinv, b, h1, h2, l1w, l1b, l2w, l2b, y2)


def kernel(x, edge_index, batch, y, W1, b1, W2, b2, W3, b3,
           lin1_W, lin1_b, lin2_W, lin2_b):
  del batch
  src = edge_index[0]
  dst = edge_index[1]
  pad = jnp.full((EPAD - E,), N, dtype=jnp.int32)
  srcp = jnp.concatenate([src, pad]).reshape(NWORK * CH, C)
  dstp = jnp.concatenate([dst, pad]).reshape(NWORK * CH, C)
  xpad = jnp.concatenate(
      [x, jnp.zeros((NP - N, D), dtype=jnp.float32)], axis=0)
  ones16 = jnp.ones((NP, 16), dtype=jnp.float32)
  zeros16 = jnp.zeros((NP, 16), dtype=jnp.float32)
  zeros32 = jnp.zeros((NP, 32), dtype=jnp.float32)

  degp = _edge_pass_16(srcp, dstp, ones16, zeros16)
  dinv, s1 = _tc_prep(degp, xpad, W1)
  p1 = _edge_pass_32(srcp, dstp, s1, zeros32)
  h1, s2 = _tc_mid(p1, s1, dinv, b1.reshape(1, 32), W2)
  p2 = _edge_pass_32(srcp, dstp, s2, zeros32)
  h2, s3 = _tc_mid(p2, s2, dinv, b2.reshape(1, 32), W3)
  p3 = _edge_pass_32(srcp, dstp, s3, zeros32)
  logits, feature, loss, acc = _tc_final(
      p3, s3, dinv, b3.reshape(1, 32), h1, h2,
      lin1_W, lin1_b.reshape(1, 128), lin2_W, lin2_b.reshape(1, NUM_CLASSES),
      y.reshape(N, 1).astype(jnp.int32))
  return logits, loss.reshape(()), acc.reshape(()), feature


# speedup vs baseline: 14.5428x; 1.2596x over previous
import functools

import jax
import jax.numpy as jnp
from jax import lax
from jax.experimental import pallas as pl
from jax.experimental.pallas import tpu as pltpu
from jax.experimental.pallas import tpu_sc as plsc

N = 10000
E = 160000
D = 256
NUM_CLASSES = 2

NCORES = 2
NSUB = 16
NWORK = NCORES * NSUB
C = 128
CH = 40
EW = C * CH
EPAD = EW * NWORK
NP = 10240
RPT = NP // NSUB


def _make_edge_pass(width, const_table):
  mesh = plsc.VectorSubcoreMesh(core_axis_name="c", subcore_axis_name="s")

  @functools.partial(
      pl.kernel,
      out_type=jax.ShapeDtypeStruct((NCORES, NP, width), jnp.float32),
      mesh=mesh,
      scratch_types=[
          pltpu.VMEM((CH, C), jnp.int32),
          pltpu.VMEM((CH, C), jnp.int32),
          pltpu.VMEM((C, width), jnp.float32),
          pltpu.VMEM((C, width), jnp.float32),
          pltpu.VMEM_SHARED((NP, width), jnp.float32),
          pltpu.SemaphoreType.DMA,
          pltpu.SemaphoreType.DMA,
      ],
      compiler_params=pltpu.CompilerParams(use_tc_tiling_on_sc=False),
  )
  def edge_pass(src_hbm, dst_hbm, table_hbm, zeros_hbm, out_hbm,
                src_v, dst_v, rows0_v, rows1_v, acc_sh, sem0, sem1):
    cid = lax.axis_index("c")
    sid = lax.axis_index("s")
    wid = cid * NSUB + sid
    pltpu.sync_copy(zeros_hbm.at[pl.ds(sid * RPT, RPT)],
                    acc_sh.at[pl.ds(sid * RPT, RPT)])
    pltpu.sync_copy(src_hbm.at[pl.ds(wid * CH, CH)], src_v)
    pltpu.sync_copy(dst_hbm.at[pl.ds(wid * CH, CH)], dst_v)
    plsc.subcore_barrier()

    if const_table:
      pltpu.sync_copy(table_hbm, rows0_v)

      def step(j, carry):
        pltpu.sync_copy(rows0_v, acc_sh.at[dst_v.at[j]], add=True)
        return carry

      lax.fori_loop(0, CH, step, 0, unroll=False)
    else:
      pltpu.async_copy(table_hbm.at[src_v.at[0]], rows0_v, sem0)
      pltpu.async_copy(table_hbm.at[src_v.at[1]], rows1_v, sem1)

      def step(jj, carry):
        j = jj * 2
        pltpu.make_async_copy(table_hbm.at[src_v.at[j]], rows0_v,
                              sem0).wait()
        pltpu.sync_copy(rows0_v, acc_sh.at[dst_v.at[j]], add=True)
        pltpu.async_copy(table_hbm.at[src_v.at[j + 2]], rows0_v, sem0)
        pltpu.make_async_copy(table_hbm.at[src_v.at[j + 1]], rows1_v,
                              sem1).wait()
        pltpu.sync_copy(rows1_v, acc_sh.at[dst_v.at[j + 1]], add=True)
        pltpu.async_copy(table_hbm.at[src_v.at[j + 3]], rows1_v, sem1)
        return carry

      lax.fori_loop(0, CH // 2 - 1, step, 0, unroll=False)
      pltpu.make_async_copy(table_hbm.at[src_v.at[CH - 2]], rows0_v,
                            sem0).wait()
      pltpu.sync_copy(rows0_v, acc_sh.at[dst_v.at[CH - 2]], add=True)
      pltpu.make_async_copy(table_hbm.at[src_v.at[CH - 1]], rows1_v,
                            sem1).wait()
      pltpu.sync_copy(rows1_v, acc_sh.at[dst_v.at[CH - 1]], add=True)

    plsc.subcore_barrier()
    pltpu.sync_copy(acc_sh.at[pl.ds(sid * RPT, RPT)],
                    out_hbm.at[cid, pl.ds(sid * RPT, RPT)])

  return edge_pass


_edge_pass_16 = _make_edge_pass(16, const_table=True)
_edge_pass_32 = _make_edge_pass(32, const_table=False)


BN = 1024


def _prep_body(deg_ref, x_ref, w1_ref, dinv_ref, s_ref):
  deg = deg_ref[0, :, 0:1] + deg_ref[1, :, 0:1] + 1.0
  dinv = 1.0 / jnp.sqrt(deg)
  dinv_ref[...] = dinv
  hw = jnp.dot(x_ref[...], w1_ref[...], preferred_element_type=jnp.float32)
  s_ref[...] = dinv * hw


def _tc_prep(degp, xpad, w1):
  return pl.pallas_call(
      _prep_body,
      grid=(NP // BN,),
      in_specs=[
          pl.BlockSpec((NCORES, BN, 16), lambda i: (0, i, 0)),
          pl.BlockSpec((BN, D), lambda i: (i, 0)),
          pl.BlockSpec((D, 32), lambda i: (0, 0)),
      ],
      out_specs=[
          pl.BlockSpec((BN, 1), lambda i: (i, 0)),
          pl.BlockSpec((BN, 32), lambda i: (i, 0)),
      ],
      out_shape=[
          jax.ShapeDtypeStruct((NP, 1), jnp.float32),
          jax.ShapeDtypeStruct((NP, 32), jnp.float32),
      ],
  )(degp, xpad, w1)


def _mid_body(p_ref, s_ref, dinv_ref, b_ref, wn_ref, h_ref, sn_ref):
  dinv = dinv_ref[...]
  h = jnp.tanh(dinv * (p_ref[0] + p_ref[1] + s_ref[...]) + b_ref[...])
  h_ref[...] = h
  sn_ref[...] = dinv * jnp.dot(h, wn_ref[...],
                               preferred_element_type=jnp.float32)


def _tc_mid(p, s, dinv, b, wn):
  return pl.pallas_call(
      _mid_body,
      grid=(NP // BN,),
      in_specs=[
          pl.BlockSpec((NCORES, BN, 32), lambda i: (0, i, 0)),
          pl.BlockSpec((BN, 32), lambda i: (i, 0)),
          pl.BlockSpec((BN, 1), lambda i: (i, 0)),
          pl.BlockSpec((1, 32), lambda i: (0, 0)),
          pl.BlockSpec((32, 32), lambda i: (0, 0)),
      ],
      out_specs=[
          pl.BlockSpec((BN, 32), lambda i: (i, 0)),
          pl.BlockSpec((BN, 32), lambda i: (i, 0)),
      ],
      out_shape=[
          jax.ShapeDtypeStruct((NP, 32), jnp.float32),
          jax.ShapeDtypeStruct((NP, 32), jnp.float32),
      ],
  )(p, s, dinv, b, wn)


BNF = 1000


def _final_body(p_ref, s_ref, dinv_ref, b_ref, h1_ref, h2_ref,
                l1w_ref, l1b_ref, l2w_ref, l2b_ref, y_ref,
                logits_ref, feat_ref, loss_ref, acc_ref):
  i = pl.program_id(0)
  dinv = dinv_ref[...]
  h3 = jnp.tanh(dinv * (p_ref[0] + p_ref[1] + s_ref[...]) + b_ref[...])
  cat = jnp.concatenate([h1_ref[...], h2_ref[...], h3], axis=1)
  hidden = jnp.dot(cat, l1w_ref[...],
                   preferred_element_type=jnp.float32) + l1b_ref[...]
  feat_ref[...] = hidden
  hr = jnp.maximum(hidden, 0.0)
  lg = jnp.dot(hr, l2w_ref[...],
               preferred_element_type=jnp.float32) + l2b_ref[...]
  m = jnp.max(lg, axis=1, keepdims=True)
  ls = lg - m - jnp.log(jnp.sum(jnp.exp(lg - m), axis=1, keepdims=True))
  logits_ref[...] = ls
  y2 = y_ref[...]
  l0 = ls[:, 0:1]
  l1 = ls[:, 1:2]
  picked = jnp.where(y2 == 0, l0, l1)
  pred = (l1 > l0).astype(jnp.int32)
  correct = (pred == y2).astype(jnp.float32)

  @pl.when(i == 0)
  def _():
    loss_ref[0, 0] = 0.0
    acc_ref[0, 0] = 0.0

  loss_ref[0, 0] += -jnp.sum(picked) / N
  acc_ref[0, 0] += jnp.sum(correct) / N


def _tc_final(p, s, dinv, b, h1, h2, l1w, l1b, l2w, l2b, y2):
  return pl.pallas_call(
      _final_body,
      grid=(N // BNF,),
      in_specs=[
          pl.BlockSpec((NCORES, BNF, 32), lambda i: (0, i, 0)),
          pl.BlockSpec((BNF, 32), lambda i: (i, 0)),
          pl.BlockSpec((BNF, 1), lambda i: (i, 0)),
          pl.BlockSpec((1, 32), lambda i: (0, 0)),
          pl.BlockSpec((BNF, 32), lambda i: (i, 0)),
          pl.BlockSpec((BNF, 32), lambda i: (i, 0)),
          pl.BlockSpec((96, 128), lambda i: (0, 0)),
          pl.BlockSpec((1, 128), lambda i: (0, 0)),
          pl.BlockSpec((128, NUM_CLASSES), lambda i: (0, 0)),
          pl.BlockSpec((1, NUM_CLASSES), lambda i: (0, 0)),
          pl.BlockSpec((BNF, 1), lambda i: (i, 0)),
      ],
      out_specs=[
          pl.BlockSpec((BNF, NUM_CLASSES), lambda i: (i, 0)),
          pl.BlockSpec((BNF, 128), lambda i: (i, 0)),
          pl.BlockSpec((1, 1), lambda i: (0, 0), memory_space=pltpu.SMEM),
          pl.BlockSpec((1, 1), lambda i: (0, 0), memory_space=pltpu.SMEM),
      ],
      out_shape=[
          jax.ShapeDtypeStruct((N, NUM_CLASSES), jnp.float32),
          jax.ShapeDtypeStruct((N, 128), jnp.float32),
          jax.ShapeDtypeStruct((1, 1), jnp.float32),
          jax.ShapeDtypeStruct((1, 1), jnp.float32),
      ],
  )(p, s, dinv, b, h1, h2, l1w, l1b, l2w, l2b, y2)


def kernel(x, edge_index, batch, y, W1, b1, W2, b2, W3, b3,
           lin1_W, lin1_b, lin2_W, lin2_b):
  del batch
  src = edge_index[0]
  dst = edge_index[1]
  pad = jnp.full((EPAD - E,), N, dtype=jnp.int32)
  srcp = jnp.concatenate([src, pad]).reshape(NWORK * CH, C)
  dstp = jnp.concatenate([dst, pad]).reshape(NWORK * CH, C)
  xpad = jnp.concatenate(
      [x, jnp.zeros((NP - N, D), dtype=jnp.float32)], axis=0)
  ones16 = jnp.ones((C, 16), dtype=jnp.float32)
  zeros16 = jnp.zeros((NP, 16), dtype=jnp.float32)
  zeros32 = jnp.zeros((NP, 32), dtype=jnp.float32)

  degp = _edge_pass_16(srcp, dstp, ones16, zeros16)
  dinv, s1 = _tc_prep(degp, xpad, W1)
  p1 = _edge_pass_32(srcp, dstp, s1, zeros32)
  h1, s2 = _tc_mid(p1, s1, dinv, b1.reshape(1, 32), W2)
  p2 = _edge_pass_32(srcp, dstp, s2, zeros32)
  h2, s3 = _tc_mid(p2, s2, dinv, b2.reshape(1, 32), W3)
  p3 = _edge_pass_32(srcp, dstp, s3, zeros32)
  logits, feature, loss, acc = _tc_final(
      p3, s3, dinv, b3.reshape(1, 32), h1, h2,
      lin1_W, lin1_b.reshape(1, 128), lin2_W, lin2_b.reshape(1, NUM_CLASSES),
      y.reshape(N, 1).astype(jnp.int32))
  return logits, loss.reshape(()), acc.reshape(()), feature
